# baseline (speedup 1.0000x reference)
import jax
import jax.numpy as jnp
from jax.experimental import pallas as pl
from jax.experimental.pallas import tpu as pltpu

_T = 256


def _moe_kernel(x_ref, wsel_ref, w1_ref, b1_ref, w2_ref, b2_ref, o_ref):
    E = wsel_ref.shape[0]
    H = b1_ref.shape[1] // E
    xt = x_ref[...]
    scores = jax.lax.dot_general(
        xt, wsel_ref[...], (((1,), (1,)), ((), ())),
        preferred_element_type=jnp.float32)
    probs = jax.nn.sigmoid(scores)
    maxv = jnp.max(probs, axis=1, keepdims=True)
    is_max = probs == maxv
    lane = jax.lax.broadcasted_iota(jnp.int32, probs.shape, 1)
    first_lane = jnp.min(jnp.where(is_max, lane, E), axis=1, keepdims=True)
    gates = jnp.where(lane == first_lane, maxv, 0.0)

    h_all = jax.lax.dot_general(
        xt, w1_ref[...], (((1,), (0,)), ((), ())),
        preferred_element_type=jnp.float32)
    h_all = jnp.maximum(h_all + b1_ref[...], 0.0)
    g_full = pltpu.repeat(gates, H, axis=1)
    hg = (h_all * g_full).astype(jnp.bfloat16)
    out = jax.lax.dot_general(
        hg, w2_ref[...], (((1,), (0,)), ((), ())),
        preferred_element_type=jnp.float32)
    out = out + jax.lax.dot_general(
        gates.astype(jnp.bfloat16), b2_ref[...], (((1,), (0,)), ((), ())),
        preferred_element_type=jnp.float32)
    o_ref[...] = out


def kernel(x, W_sel, W1, b1, W2, b2):
    B, S, D = x.shape
    E, _, H = W1.shape
    xb = x.reshape(S, D).astype(jnp.bfloat16)
    wsel_b = W_sel.astype(jnp.bfloat16)
    w1i = jnp.transpose(W1, (1, 2, 0)).reshape(D, H * E).astype(jnp.bfloat16)
    b1i = jnp.transpose(b1, (1, 0)).reshape(1, H * E)
    w2i = jnp.transpose(W2, (1, 0, 2)).reshape(H * E, D).astype(jnp.bfloat16)
    b2b = b2.astype(jnp.bfloat16)
    out = pl.pallas_call(
        _moe_kernel,
        grid=(S // _T,),
        in_specs=[
            pl.BlockSpec((_T, D), lambda i: (i, 0)),
            pl.BlockSpec((E, D), lambda i: (0, 0)),
            pl.BlockSpec((D, H * E), lambda i: (0, 0)),
            pl.BlockSpec((1, H * E), lambda i: (0, 0)),
            pl.BlockSpec((H * E, D), lambda i: (0, 0)),
            pl.BlockSpec((E, D), lambda i: (0, 0)),
        ],
        out_specs=pl.BlockSpec((_T, D), lambda i: (i, 0)),
        out_shape=jax.ShapeDtypeStruct((S, D), jnp.float32),
        compiler_params=pltpu.CompilerParams(
            dimension_semantics=("parallel",)),
    )(xb, wsel_b, w1i, b1i, w2i, b2b)
    return out.reshape(B, S, D)


# speedup vs baseline: 1.5177x; 1.5177x over previous
import jax
import jax.numpy as jnp
from jax.experimental import pallas as pl
from jax.experimental.pallas import tpu as pltpu

_T = 256
_CORES = 2


def _moe_kernel(x_ref, wsel_ref, w1_ref, b1_ref, w2_ref, b2_ref, o_ref,
                w1s, w2s):
    E = wsel_ref.shape[0]

    @pl.when(pl.program_id(1) == 0)
    def _():
        w1s[...] = w1_ref[...].astype(jnp.bfloat16)
        w2s[...] = w2_ref[...].astype(jnp.bfloat16)

    xt = x_ref[...].astype(jnp.bfloat16)
    scores = jax.lax.dot_general(
        xt, wsel_ref[...].astype(jnp.bfloat16), (((1,), (1,)), ((), ())),
        preferred_element_type=jnp.float32)
    probs = jax.nn.sigmoid(scores)
    maxv = jnp.max(probs, axis=1, keepdims=True)
    is_max = probs == maxv
    lane = jax.lax.broadcasted_iota(jnp.int32, probs.shape, 1)
    first_lane = jnp.min(jnp.where(is_max, lane, E), axis=1, keepdims=True)
    gates = jnp.where(lane == first_lane, maxv, 0.0)

    acc = jax.lax.dot_general(
        gates.astype(jnp.bfloat16), b2_ref[...].astype(jnp.bfloat16),
        (((1,), (0,)), ((), ())), preferred_element_type=jnp.float32)
    for e in range(E):
        h = jax.lax.dot_general(
            xt, w1s[e], (((1,), (0,)), ((), ())),
            preferred_element_type=jnp.float32)
        h = jnp.maximum(h + b1_ref[e][None, :], 0.0)
        hg = (h * gates[:, e:e + 1]).astype(jnp.bfloat16)
        acc = acc + jax.lax.dot_general(
            hg, w2s[e], (((1,), (0,)), ((), ())),
            preferred_element_type=jnp.float32)
    o_ref[...] = acc


def kernel(x, W_sel, W1, b1, W2, b2):
    B, S, D = x.shape
    E, _, H = W1.shape
    xf = x.reshape(S, D)
    steps = S // _T // _CORES
    out = pl.pallas_call(
        _moe_kernel,
        grid=(_CORES, steps),
        in_specs=[
            pl.BlockSpec((_T, D), lambda i, j: (i * steps + j, 0)),
            pl.BlockSpec((E, D), lambda i, j: (0, 0)),
            pl.BlockSpec((E, D, H), lambda i, j: (0, 0, 0)),
            pl.BlockSpec((E, H), lambda i, j: (0, 0)),
            pl.BlockSpec((E, H, D), lambda i, j: (0, 0, 0)),
            pl.BlockSpec((E, D), lambda i, j: (0, 0)),
        ],
        out_specs=pl.BlockSpec((_T, D), lambda i, j: (i * steps + j, 0)),
        out_shape=jax.ShapeDtypeStruct((S, D), jnp.float32),
        scratch_shapes=[
            pltpu.VMEM((E, D, H), jnp.bfloat16),
            pltpu.VMEM((E, H, D), jnp.bfloat16),
        ],
        compiler_params=pltpu.CompilerParams(
            dimension_semantics=("parallel", "arbitrary")),
    )(xf, W_sel, W1, b1, W2, b2)
    return out.reshape(B, S, D)


# speedup vs baseline: 1.6043x; 1.0570x over previous
import jax
import jax.numpy as jnp
from jax.experimental import pallas as pl
from jax.experimental.pallas import tpu as pltpu

_T = 256


def _moe_kernel(x_ref, wsel_ref, w1_ref, b1_ref, w2_ref, b2_ref, o_ref,
                w1b, w2b, hg):
    E, D = wsel_ref.shape
    H = b1_ref.shape[1]

    @pl.when(pl.program_id(0) == 0)
    def _convert():
        w1b[...] = w1_ref[...].astype(jnp.bfloat16)
        for e in range(E):
            w2b[e * H:(e + 1) * H, :] = w2_ref[e].astype(jnp.bfloat16)

    xt = x_ref[...].astype(jnp.bfloat16)
    scores = jax.lax.dot_general(
        xt, wsel_ref[...].astype(jnp.bfloat16), (((1,), (1,)), ((), ())),
        preferred_element_type=jnp.float32)
    probs = jax.nn.sigmoid(scores)
    maxv = jnp.max(probs, axis=1, keepdims=True)
    is_max = probs == maxv
    lane = jax.lax.broadcasted_iota(jnp.int32, probs.shape, 1)
    first_lane = jnp.min(jnp.where(is_max, lane, E), axis=1, keepdims=True)
    gates = jnp.where(lane == first_lane, maxv, 0.0)

    for e in range(E):
        h = jax.lax.dot_general(
            xt, w1b[e], (((1,), (0,)), ((), ())),
            preferred_element_type=jnp.float32)
        h = jnp.maximum(h + b1_ref[e][None, :], 0.0)
        hg[:, e * H:(e + 1) * H] = (h * gates[:, e:e + 1]).astype(jnp.bfloat16)

    out = jax.lax.dot_general(
        hg[...], w2b[...], (((1,), (0,)), ((), ())),
        preferred_element_type=jnp.float32)
    o_ref[...] = out + jax.lax.dot_general(
        gates.astype(jnp.bfloat16), b2_ref[...].astype(jnp.bfloat16),
        (((1,), (0,)), ((), ())), preferred_element_type=jnp.float32)


def kernel(x, W_sel, W1, b1, W2, b2):
    B, S, D = x.shape
    E, _, H = W1.shape
    xf = x.reshape(S, D)
    out = pl.pallas_call(
        _moe_kernel,
        grid=(S // _T,),
        in_specs=[
            pl.BlockSpec((_T, D), lambda i: (i, 0)),
            pl.BlockSpec((E, D), lambda i: (0, 0)),
            pl.BlockSpec((E, D, H), lambda i: (0, 0, 0)),
            pl.BlockSpec((E, H), lambda i: (0, 0)),
            pl.BlockSpec((E, H, D), lambda i: (0, 0, 0)),
            pl.BlockSpec((E, D), lambda i: (0, 0)),
        ],
        out_specs=pl.BlockSpec((_T, D), lambda i: (i, 0)),
        out_shape=jax.ShapeDtypeStruct((S, D), jnp.float32),
        scratch_shapes=[
            pltpu.VMEM((E, D, H), jnp.bfloat16),
            pltpu.VMEM((E * H, D), jnp.bfloat16),
            pltpu.VMEM((_T, E * H), jnp.bfloat16),
        ],
        compiler_params=pltpu.CompilerParams(
            dimension_semantics=("arbitrary",)),
    )(xf, W_sel, W1, b1, W2, b2)
    return out.reshape(B, S, D)
